# Initial kernel scaffold; baseline (speedup 1.0000x reference)
import jax
import jax.numpy as jnp
from jax.experimental import pallas as pl


_BS = 256


def _add_body(x_ref, pos_ref, out_ref):
    out_ref[...] = x_ref[...] + pos_ref[:, None, :]


def kernel(x, pos_embedding):
    seq, batch, d = x.shape
    grid = (seq // _BS,)
    return pl.pallas_call(
        _add_body,
        grid=grid,
        in_specs=[
            pl.BlockSpec((_BS, batch, d), lambda i: (i, 0, 0)),
            pl.BlockSpec((_BS, d), lambda i: (i, 0)),
        ],
        out_specs=pl.BlockSpec((_BS, batch, d), lambda i: (i, 0, 0)),
        out_shape=jax.ShapeDtypeStruct((seq, batch, d), x.dtype),
    )(x, pos_embedding[:seq])


# speedup vs baseline: 1.6657x; 1.6657x over previous
import jax
import jax.numpy as jnp
from jax.experimental import pallas as pl


_BS = 256


def _add_body(x_ref, pos_ref, out_ref):
    out_ref[...] = x_ref[...] + pos_ref[...][:, None, :]


def kernel(x, pos_embedding):
    seq, batch, d = x.shape
    grid = (seq // _BS,)
    return pl.pallas_call(
        _add_body,
        grid=grid,
        in_specs=[
            pl.BlockSpec((_BS, batch, d), lambda i: (i, 0, 0)),
            pl.BlockSpec((_BS, d), lambda i: (i, 0)),
        ],
        out_specs=pl.BlockSpec((_BS, batch, d), lambda i: (i, 0, 0)),
        out_shape=jax.ShapeDtypeStruct((seq, batch, d), x.dtype),
    )(x, pos_embedding[:seq])


# speedup vs baseline: 1.7008x; 1.0211x over previous
import jax
import jax.numpy as jnp
from jax.experimental import pallas as pl


_BS = 512


def _add_body(x_ref, pos_ref, out_ref):
    out_ref[...] = x_ref[...] + pos_ref[...][:, None, :]


def kernel(x, pos_embedding):
    seq, batch, d = x.shape
    grid = (seq // _BS,)
    return pl.pallas_call(
        _add_body,
        grid=grid,
        in_specs=[
            pl.BlockSpec((_BS, batch, d), lambda i: (i, 0, 0)),
            pl.BlockSpec((_BS, d), lambda i: (i, 0)),
        ],
        out_specs=pl.BlockSpec((_BS, batch, d), lambda i: (i, 0, 0)),
        out_shape=jax.ShapeDtypeStruct((seq, batch, d), x.dtype),
    )(x, pos_embedding[:seq])
